# Initial kernel scaffold; baseline (speedup 1.0000x reference)
import jax
import jax.numpy as jnp
from jax.experimental import pallas as pl


def kernel(f_in, pos, edge_src, edge_dst, max_radius, W1, W2):
    raise NotImplementedError("write your pallas kernel here")


# speedup vs baseline: 2.4938x; 2.4938x over previous
import functools

import numpy as np
import jax
import jax.numpy as jnp
from jax import lax
from jax.experimental import pallas as pl
from jax.experimental.pallas import tpu as pltpu
from jax.experimental.pallas import tpu_sc as plsc

NUM_BASIS = 10
MUL = 16
N_NODES = 10000
N_EDGES = 160000
OUT_DIM = 144
NC, NS = 2, 16
NW = NC * NS
CH = 128
ROWS = N_EDGES // CH
RPC = ROWS // NC
NPT = N_NODES // NS
BE = 2000


def _static_mats():
    l = np.zeros(OUT_DIM, np.int32)
    v = np.zeros(OUT_DIM, np.int32)
    j = np.zeros(OUT_DIM, np.int32)
    for c in range(OUT_DIM):
        if c < 16:
            l[c], v[c], j[c] = 0, c, 0
        elif c < 64:
            l[c], v[c], j[c] = 1, (c - 16) // 3, (c - 16) % 3
        else:
            l[c], v[c], j[c] = 2, (c - 64) // 5, (c - 64) % 5
    colmap = l * MUL + v
    RH = np.zeros((MUL, MUL * MUL), np.float32)
    RX = np.zeros((MUL, MUL * MUL), np.float32)
    for k in range(MUL):
        for u in range(MUL):
            RH[k, k * MUL + u] = 1.0
            RX[u, k * MUL + u] = 1.0
    jg = np.where(l == 0, 0, np.where(l == 1, 1 + j, 4 + j))
    Q = np.zeros((16, OUT_DIM), np.float32)
    for c in range(OUT_DIM):
        Q[jg[c], c] = 1.0
    return colmap, RH, RX, Q


_COLMAP, _RH, _RX, _Q = _static_mats()


def _sc_gather(t1, t2, src2d, dst2d):
    mesh = plsc.VectorSubcoreMesh(core_axis_name="c", subcore_axis_name="s")

    @functools.partial(
        pl.kernel,
        out_type=(jax.ShapeDtypeStruct((N_EDGES, 32), jnp.float32),
                  jax.ShapeDtypeStruct((N_EDGES, 16), jnp.float32)),
        mesh=mesh,
        scratch_types=[
            pltpu.VMEM((CH,), jnp.int32),
            pltpu.VMEM((CH,), jnp.int32),
            pltpu.VMEM((CH, 32), jnp.float32),
            pltpu.VMEM((CH, 16), jnp.float32),
            pltpu.SemaphoreType.DMA,
            pltpu.SemaphoreType.DMA,
        ],
        compiler_params=pltpu.CompilerParams(use_tc_tiling_on_sc=False),
    )
    def k(t1h, t2h, srch, dsth, out1, out2, idx1, idx2, buf1, buf2, sem1, sem2):
        w = lax.axis_index("s") * NC + lax.axis_index("c")

        def body(i, carry):
            row = w + i * NW

            @pl.when(row < ROWS)
            def _():
                pltpu.sync_copy(srch.at[row], idx1)
                pltpu.sync_copy(dsth.at[row], idx2)
                cp1 = pltpu.async_copy(t1h.at[idx1], buf1, sem1)
                cp2 = pltpu.async_copy(t2h.at[idx2], buf2, sem2)
                cp1.wait()
                cp2.wait()
                pltpu.sync_copy(buf1, out1.at[pl.ds(row * CH, CH)])
                pltpu.sync_copy(buf2, out2.at[pl.ds(row * CH, CH)])

            return carry

        lax.fori_loop(0, (ROWS + NW - 1) // NW, body, 0)

    return k(t1, t2, src2d, dst2d)


def _tc_msg(srcg, dstg, w1p, w2p, rh, rx, q, prm):
    grid = N_EDGES // BE

    def body(prm_ref, w1_ref, w2_ref, rh_ref, rx_ref, q_ref, s_ref, d_ref, o_ref):
        sg = s_ref[...]
        dg = d_ref[...]
        x = sg[:, 0:16]
        ev = dg[:, 0:3] - sg[:, 16:19]
        d2 = jnp.sum(ev * ev, axis=1, keepdims=True)
        dist = jnp.sqrt(d2 + 1e-9)
        u = ev / dist
        ux = u[:, 0:1]
        uy = u[:, 1:2]
        uz = u[:, 2:3]
        vals = prm_ref[0:1, :]
        step = prm_ref[1:2, 0:1]
        diff = (dist - vals) / step

        def sus(t):
            return jnp.where(t > 0.0, jnp.exp(-1.0 / jnp.where(t > 0.0, t, 1.0)), 0.0)

        soft = (1.14136 * np.exp(2.0)) * sus(diff + 1.0) * sus(1.0 - diff)
        col = lax.broadcasted_iota(jnp.int32, (BE, 16), 1)
        soft = jnp.where(col < NUM_BASIS, soft, 0.0)
        h = np.sqrt(2.0) * jax.nn.relu(
            jnp.dot(soft, w1_ref[...], preferred_element_type=jnp.float32))
        g = (jnp.dot(h, rh_ref[...], preferred_element_type=jnp.float32)
             * jnp.dot(x, rx_ref[...], preferred_element_type=jnp.float32))
        m = jnp.dot(g, w2_ref[...], preferred_element_type=jnp.float32)
        c1 = np.sqrt(3.0)
        c2 = np.sqrt(15.0)
        shc = (jnp.ones_like(ux), c1 * ux, c1 * uy, c1 * uz,
               c2 * ux * uz, c2 * ux * uy,
               (np.sqrt(5.0) / 2.0) * (3.0 * uy * uy - 1.0),
               c2 * uy * uz, (c2 / 2.0) * (uz * uz - ux * ux))
        s = shc[0] * q_ref[0:1, :]
        for t in range(1, 9):
            s = s + shc[t] * q_ref[t:t + 1, :]
        o_ref[...] = m * s

    small = lambda shp: pl.BlockSpec(shp, lambda i: (0, 0))
    return pl.pallas_call(
        body,
        grid=(grid,),
        in_specs=[
            small((8, 16)),
            small((16, 16)),
            small((256, OUT_DIM)),
            small((16, 256)),
            small((16, 256)),
            small((16, OUT_DIM)),
            pl.BlockSpec((BE, 32), lambda i: (i, 0)),
            pl.BlockSpec((BE, 16), lambda i: (i, 0)),
        ],
        out_specs=pl.BlockSpec((BE, OUT_DIM), lambda i: (i, 0)),
        out_shape=jax.ShapeDtypeStruct((N_EDGES, OUT_DIM), jnp.float32),
    )(prm, w1p, w2p, rh, rx, q, srcg, dstg)


def _sc_scatter(msg, dst2d):
    mesh = plsc.VectorSubcoreMesh(core_axis_name="c", subcore_axis_name="s")

    @functools.partial(
        pl.kernel,
        out_type=jax.ShapeDtypeStruct((NC * N_NODES, OUT_DIM), jnp.float32),
        mesh=mesh,
        scratch_types=[
            pltpu.VMEM((1, CH), jnp.int32),
            pltpu.VMEM((CH, OUT_DIM), jnp.float32),
            pltpu.VMEM((125, OUT_DIM), jnp.float32),
            pltpu.VMEM_SHARED((N_NODES, OUT_DIM), jnp.float32),
        ],
        compiler_params=pltpu.CompilerParams(use_tc_tiling_on_sc=False),
    )
    def k(msgh, dsth, outh, idxb, mbuf, zbuf, acc):
        c = lax.axis_index("c")
        s = lax.axis_index("s")
        zeros16 = jnp.zeros((16,), jnp.float32)

        def zrow(i, carry):
            def zcol(t, carry2):
                zbuf[i, pl.ds(t * 16, 16)] = zeros16
                return carry2

            return lax.fori_loop(0, OUT_DIM // 16, zcol, carry)

        lax.fori_loop(0, 125, zrow, 0)

        def zcp(t, carry):
            pltpu.sync_copy(zbuf, acc.at[pl.ds(s * NPT + t * 125, 125)])
            return carry

        lax.fori_loop(0, NPT // 125, zcp, 0)
        plsc.subcore_barrier()

        def body(i, carry):
            lrow = s + i * NS

            @pl.when(lrow < RPC)
            def _():
                row = c * RPC + lrow
                pltpu.sync_copy(dsth.at[row], idxb.at[0])
                pltpu.sync_copy(msgh.at[pl.ds(row * CH, CH)], mbuf)
                pltpu.sync_copy(mbuf, acc.at[idxb.at[0]], add=True)

            return carry

        lax.fori_loop(0, (RPC + NS - 1) // NS, body, 0)
        plsc.subcore_barrier()
        pltpu.sync_copy(acc.at[pl.ds(s * NPT, NPT)],
                        outh.at[pl.ds(c * N_NODES + s * NPT, NPT)])

    return k(msg, dst2d)


def _tc_add(partials):
    bn = 2000

    def body(a_ref, b_ref, o_ref):
        o_ref[...] = a_ref[...] + b_ref[...]

    return pl.pallas_call(
        body,
        grid=(N_NODES // bn,),
        in_specs=[
            pl.BlockSpec((bn, OUT_DIM), lambda i: (i, 0)),
            pl.BlockSpec((bn, OUT_DIM), lambda i: (i + N_NODES // bn, 0)),
        ],
        out_specs=pl.BlockSpec((bn, OUT_DIM), lambda i: (i, 0)),
        out_shape=jax.ShapeDtypeStruct((N_NODES, OUT_DIM), jnp.float32),
    )(partials, partials)


def kernel(f_in, pos, edge_src, edge_dst, max_radius, W1, W2):
    f_in = f_in.astype(jnp.float32)
    pos = pos.astype(jnp.float32)
    t1 = jnp.concatenate([f_in, pos, jnp.zeros((N_NODES, 13), jnp.float32)], axis=1)
    t2 = jnp.concatenate([pos, jnp.zeros((N_NODES, 13), jnp.float32)], axis=1)
    src2d = edge_src.astype(jnp.int32).reshape(ROWS, CH)
    dst2d = edge_dst.astype(jnp.int32).reshape(ROWS, CH)

    w2r = W2.astype(jnp.float32).reshape(MUL, 3, MUL, MUL)
    w2kl = jnp.transpose(w2r, (0, 2, 1, 3)).reshape(MUL * MUL, 3 * MUL)
    w2p = jnp.take(w2kl, jnp.asarray(_COLMAP), axis=1) * (1.0 / 64.0)
    w1p = jnp.zeros((16, 16), jnp.float32).at[:NUM_BASIS].set(W1.astype(jnp.float32))

    mr = jnp.asarray(max_radius, jnp.float32)
    step = mr / (NUM_BASIS + 1)
    vals = step * jnp.arange(1, NUM_BASIS + 1, dtype=jnp.float32)
    prm = (jnp.zeros((8, 16), jnp.float32)
           .at[0, :NUM_BASIS].set(vals)
           .at[1, :].set(step))

    srcg, dstg = _sc_gather(t1, t2, src2d, dst2d)
    msg = _tc_msg(srcg, dstg, w1p, w2p,
                  jnp.asarray(_RH), jnp.asarray(_RX), jnp.asarray(_Q), prm)
    partials = _sc_scatter(msg, dst2d)
    return _tc_add(partials)
